# baseline (speedup 1.0000x reference)
import functools

import jax
import jax.numpy as jnp
from jax import lax
from jax.experimental import pallas as pl
from jax.experimental.pallas import tpu as pltpu
from jax.experimental.pallas import tpu_sc as plsc

N = 8388608
ROWS = 65536
COLS = 128
BLOCK_ROWS = 8192
GRID = ROWS // BLOCK_ROWS

NC = 2
NS = 16
L = 16
NW = NC * NS
PER_W = N // NW
SC_CHUNK = 16384
SC_NCH = PER_W // SC_CHUNK


def _sc_count_body(t_hbm, out_hbm, buf0, buf1, acc_v, sem0, sem1):
    c = lax.axis_index("c")
    s = lax.axis_index("s")
    wid = s * NC + c
    base = wid * PER_W

    bufs = (buf0, buf1)
    sems = (sem0, sem1)

    copies = [None, None]
    copies[0] = pltpu.async_copy(
        t_hbm.at[pl.ds(base, SC_CHUNK)], buf0, sem0)

    acc = jnp.zeros((L,), jnp.float32)
    for g in range(SC_NCH):
        cur = g % 2
        nxt = (g + 1) % 2
        if g + 1 < SC_NCH:
            copies[nxt] = pltpu.async_copy(
                t_hbm.at[pl.ds(base + (g + 1) * SC_CHUNK, SC_CHUNK)],
                bufs[nxt], sems[nxt])
        copies[cur].wait()
        buf = bufs[cur]

        def inner(j, a):
            return a + buf[pl.ds(j * L, L)]

        acc = lax.fori_loop(0, SC_CHUNK // L, inner, acc, unroll=16)

    acc_v[...] = acc
    pltpu.sync_copy(acc_v, out_hbm.at[wid])


def _sc_count(target):
    run = pl.kernel(
        _sc_count_body,
        out_type=jax.ShapeDtypeStruct((NW, L), jnp.float32),
        mesh=plsc.VectorSubcoreMesh(core_axis_name="c", subcore_axis_name="s"),
        scratch_types=[
            pltpu.VMEM((SC_CHUNK,), jnp.float32),
            pltpu.VMEM((SC_CHUNK,), jnp.float32),
            pltpu.VMEM((L,), jnp.float32),
            pltpu.SemaphoreType.DMA,
            pltpu.SemaphoreType.DMA,
        ],
    )
    return run(target)


def _fused_body(p_ref, t_ref, mix_ref):
    i = pl.program_id(0)

    p = p_ref[...]
    t = t_ref[...]

    LN2 = 0.6931471805599453
    NLOG2E = -1.4426950408889634
    y = jnp.minimum(p * NLOG2E, 126.0)
    e = jnp.exp2(y)
    sp = LN2 * jnp.log2(1.0 + e)
    u = LN2 * jnp.log2(1.0 + sp)
    mix = jnp.where(t >= 0.5, -100.0, u)

    mix_part = jnp.sum(mix.reshape(BLOCK_ROWS // 8, 8, COLS), axis=0)

    @pl.when(i == 0)
    def _():
        mix_ref[...] = jnp.zeros_like(mix_ref)

    mix_ref[...] += mix_part


def _tc_mix(pred, target):
    p2 = pred.reshape(ROWS, COLS)
    t2 = target.reshape(ROWS, COLS)
    return pl.pallas_call(
        _fused_body,
        grid=(GRID,),
        in_specs=[
            pl.BlockSpec((BLOCK_ROWS, COLS), lambda i: (i, 0)),
            pl.BlockSpec((BLOCK_ROWS, COLS), lambda i: (i, 0)),
        ],
        out_specs=pl.BlockSpec((8, COLS), lambda i: (0, 0)),
        out_shape=jax.ShapeDtypeStruct((8, COLS), jnp.float32),
    )(p2, t2)


def kernel(pred, target):
    counts = _sc_count(target)
    mix_acc = _tc_mix(pred, target)

    s_t = jnp.sum(counts)
    s_mix = jnp.sum(mix_acc)

    a = jnp.float32(N) - s_t
    b = s_t
    w = a / b
    return -(w * s_mix) / jnp.float32(N)
